# baseline (speedup 1.0000x reference)
import jax
import jax.numpy as jnp
from jax.experimental import pallas as pl
from jax.experimental.pallas import tpu as pltpu

_NBUF = 4
_CHUNK = 256


def _dma_kernel(emb_ref, out_ref, buf, in_sems, out_sems):
    rows = out_ref.shape[0]
    n = rows // _CHUNK

    def in_copy(g):
        return pltpu.make_async_copy(
            emb_ref.at[pl.ds(g * _CHUNK, _CHUNK), :],
            buf.at[g % _NBUF],
            in_sems.at[g % _NBUF],
        )

    def out_copy(g):
        return pltpu.make_async_copy(
            buf.at[g % _NBUF],
            out_ref.at[pl.ds(g * _CHUNK, _CHUNK), :],
            out_sems.at[g % _NBUF],
        )

    for g in range(min(_NBUF, n)):
        in_copy(g).start()
    for g in range(n):
        in_copy(g).wait()
        out_copy(g).start()
        if g + _NBUF < n:
            out_copy(g).wait()
            in_copy(g + _NBUF).start()
    for g in range(max(0, n - _NBUF), n):
        out_copy(g).wait()


def kernel(x, emb_weight):
    seq_len = x.shape[1]
    dim = emb_weight.shape[1]
    return pl.pallas_call(
        _dma_kernel,
        in_specs=[pl.BlockSpec(memory_space=pl.ANY)],
        out_specs=pl.BlockSpec(memory_space=pl.ANY),
        out_shape=jax.ShapeDtypeStruct((seq_len, dim), emb_weight.dtype),
        scratch_shapes=[
            pltpu.VMEM((_NBUF, _CHUNK, dim), jnp.float32),
            pltpu.SemaphoreType.DMA((_NBUF,)),
            pltpu.SemaphoreType.DMA((_NBUF,)),
        ],
    )(emb_weight)
